# speedup vs baseline: 7.8685x; 7.8685x over previous
import jax
import jax.numpy as jnp
from jax.experimental import pallas as pl
from jax.experimental.pallas import tpu as pltpu

B, IMG, PATCH = 32, 224, 16
D_MODEL, DEPTH = 384, 24
D_INNER, D_STATE, DT_RANK, D_CONV = 768, 16, 24, 4
L = (IMG // PATCH) ** 2
NP = IMG // PATCH
NCLS = 1000
EPS = 1e-5
PDIM = 3 * PATCH * PATCH


def _ln(x, w, b):
    mu = jnp.mean(x, axis=-1, keepdims=True)
    xc = x - mu
    var = jnp.mean(xc * xc, axis=-1, keepdims=True)
    return xc * jax.lax.rsqrt(var + EPS) * w + b


def _silu(x):
    return x / (1.0 + jnp.exp(-x))


def _softplus(x):
    return jnp.maximum(x, 0.0) + jnp.log(1.0 + jnp.exp(-jnp.abs(x)))


def _vim_body(patches_ref, pwT_ref, pb_ref, pos_ref, WinT_ref, cwT_ref, cb_ref,
              WxpT_ref, WdtT_ref, bdt_ref, AT_ref, Dp_ref, WoutT_ref,
              lnw_ref, lnb_ref, nfw_ref, nfb_ref,
              pooled_ref,
              res_scr, hid_scr, dA_scr, bu_scr):
    d = pl.program_id(1)

    @pl.when(d == 0)
    def _():
        p = patches_ref[0]
        h0 = jnp.dot(p, pwT_ref[...], preferred_element_type=jnp.float32)
        hid_scr[...] = h0 + pb_ref[...] + pos_ref[...]
        res_scr[...] = jnp.zeros_like(res_scr)

    res = res_scr[...] + hid_scr[...]
    res_scr[...] = res
    x = _ln(res, lnw_ref[0], lnb_ref[0])

    xz = jnp.dot(x, WinT_ref[0], preferred_element_type=jnp.float32)
    xc0 = xz[:, :D_INNER]
    z = xz[:, D_INNER:]

    cw = cwT_ref[0]
    xp = jnp.concatenate([jnp.zeros((D_CONV - 1, D_INNER), jnp.float32), xc0],
                         axis=0)
    acc = xp[0:L, :] * cw[0:1, :]
    for k in range(1, D_CONV):
        acc = acc + xp[k:k + L, :] * cw[k:k + 1, :]
    xc = _silu(acc + cb_ref[0])

    proj = jnp.dot(xc, WxpT_ref[0], preferred_element_type=jnp.float32)
    dt = proj[:, :DT_RANK]
    Bs = proj[:, DT_RANK:DT_RANK + D_STATE]
    Cs = proj[:, DT_RANK + D_STATE:]

    delta = _softplus(
        jnp.dot(dt, WdtT_ref[0], preferred_element_type=jnp.float32)
        + bdt_ref[0])

    A = -jnp.exp(AT_ref[0])
    dA_scr[...] = jnp.exp(delta[:, None, :] * A[None, :, :])
    bu_scr[...] = (delta * xc)[:, None, :] * Bs[:, :, None]

    def step(t, h):
        h = dA_scr[t] * h + bu_scr[t]
        dA_scr[t] = h
        return h

    jax.lax.fori_loop(0, L, step, jnp.zeros((D_STATE, D_INNER), jnp.float32))

    H = dA_scr[...]
    ys = jnp.sum(H * Cs[:, :, None], axis=1)
    y = ys + xc * Dp_ref[0]
    y = y * _silu(z)
    out = jnp.dot(y, WoutT_ref[0], preferred_element_type=jnp.float32)
    hid_scr[...] = out

    @pl.when(d == DEPTH - 1)
    def _():
        resf = res + out
        hf = _ln(resf, nfw_ref[...], nfb_ref[...])
        pooled_ref[0] = jnp.mean(hf, axis=0, keepdims=True)


def _head_body(p_ref, w_ref, b_ref, o_ref):
    o_ref[...] = (jnp.dot(p_ref[...], w_ref[...],
                          preferred_element_type=jnp.float32) + b_ref[...])


def kernel(x, patch_w, patch_b, pos_embed, W_in, conv_w, conv_b, W_xp, W_dt,
           b_dt, A_log, D_p, W_out, ln_w, ln_b, normf_w, normf_b, head_w,
           head_b):
    f32 = jnp.float32
    patches = x.reshape(B, 3, NP, PATCH, NP, PATCH).transpose(0, 2, 4, 1, 3, 5)
    patches = patches.reshape(B, L, PDIM)
    pwT = patch_w.reshape(D_MODEL, PDIM).T
    pb = patch_b.reshape(1, D_MODEL)
    pos = pos_embed.reshape(L, D_MODEL)
    WinT = jnp.swapaxes(W_in, 1, 2)
    cwT = jnp.swapaxes(conv_w, 1, 2)
    cb = conv_b.reshape(DEPTH, 1, D_INNER)
    WxpT = jnp.swapaxes(W_xp, 1, 2)
    WdtT = jnp.swapaxes(W_dt, 1, 2)
    bdt = b_dt.reshape(DEPTH, 1, D_INNER)
    AT = jnp.swapaxes(A_log, 1, 2)
    Dp = D_p.reshape(DEPTH, 1, D_INNER)
    WoutT = jnp.swapaxes(W_out, 1, 2)
    lnw = ln_w.reshape(DEPTH, 1, D_MODEL)
    lnb = ln_b.reshape(DEPTH, 1, D_MODEL)
    nfw = normf_w.reshape(1, D_MODEL)
    nfb = normf_b.reshape(1, D_MODEL)

    bcast = lambda i, d: (0, 0)
    perb = lambda i, d: (i, 0, 0)
    perd = lambda i, d: (d, 0, 0)

    pooled = pl.pallas_call(
        _vim_body,
        grid=(B, DEPTH),
        in_specs=[
            pl.BlockSpec((1, L, PDIM), perb),
            pl.BlockSpec((PDIM, D_MODEL), bcast),
            pl.BlockSpec((1, D_MODEL), bcast),
            pl.BlockSpec((L, D_MODEL), bcast),
            pl.BlockSpec((1, D_MODEL, 2 * D_INNER), perd),
            pl.BlockSpec((1, D_CONV, D_INNER), perd),
            pl.BlockSpec((1, 1, D_INNER), perd),
            pl.BlockSpec((1, D_INNER, DT_RANK + 2 * D_STATE), perd),
            pl.BlockSpec((1, DT_RANK, D_INNER), perd),
            pl.BlockSpec((1, 1, D_INNER), perd),
            pl.BlockSpec((1, D_STATE, D_INNER), perd),
            pl.BlockSpec((1, 1, D_INNER), perd),
            pl.BlockSpec((1, D_INNER, D_MODEL), perd),
            pl.BlockSpec((1, 1, D_MODEL), perd),
            pl.BlockSpec((1, 1, D_MODEL), perd),
            pl.BlockSpec((1, D_MODEL), bcast),
            pl.BlockSpec((1, D_MODEL), bcast),
        ],
        out_specs=pl.BlockSpec((1, 1, D_MODEL), perb),
        out_shape=jax.ShapeDtypeStruct((B, 1, D_MODEL), f32),
        scratch_shapes=[
            pltpu.VMEM((L, D_MODEL), f32),
            pltpu.VMEM((L, D_MODEL), f32),
            pltpu.VMEM((L, D_STATE, D_INNER), f32),
            pltpu.VMEM((L, D_STATE, D_INNER), f32),
        ],
        compiler_params=pltpu.CompilerParams(
            dimension_semantics=("parallel", "arbitrary"),
        ),
    )(patches, pwT, pb, pos, WinT, cwT, cb, WxpT, WdtT, bdt, AT, Dp, WoutT,
      lnw, lnb, nfw, nfb)

    logits = pl.pallas_call(
        _head_body,
        out_shape=jax.ShapeDtypeStruct((B, NCLS), f32),
    )(pooled.reshape(B, D_MODEL), head_w.T, head_b.reshape(1, NCLS))
    return logits


# speedup vs baseline: 8.6859x; 1.1039x over previous
import jax
import jax.numpy as jnp
from jax.experimental import pallas as pl
from jax.experimental.pallas import tpu as pltpu

B, IMG, PATCH = 32, 224, 16
D_MODEL, DEPTH = 384, 24
D_INNER, D_STATE, DT_RANK, D_CONV = 768, 16, 24, 4
L = (IMG // PATCH) ** 2
NP = IMG // PATCH
NCLS = 1000
EPS = 1e-5
PDIM = 3 * PATCH * PATCH


def _ln(x, w, b):
    mu = jnp.mean(x, axis=-1, keepdims=True)
    xc = x - mu
    var = jnp.mean(xc * xc, axis=-1, keepdims=True)
    return xc * jax.lax.rsqrt(var + EPS) * w + b


def _silu(x):
    return x / (1.0 + jnp.exp(-x))


def _softplus(x):
    return jnp.maximum(x, 0.0) + jnp.log(1.0 + jnp.exp(-jnp.abs(x)))


def _vim_body(patches_ref, pwT_ref, pb_ref, pos_ref, WinT_ref, cwT_ref, cb_ref,
              WxpT_ref, WdtT_ref, bdt_ref, AT_ref, Dp_ref, WoutT_ref,
              lnw_ref, lnb_ref, nfw_ref, nfb_ref,
              pooled_ref,
              res_scr, hid_scr, dA_scr, bu_scr, h_scr, fence_sem):
    d = pl.program_id(1)

    @pl.when(d == 0)
    def _():
        p = patches_ref[0]
        h0 = jnp.dot(p, pwT_ref[...], preferred_element_type=jnp.float32)
        hid_scr[...] = h0 + pb_ref[...] + pos_ref[...]
        res_scr[...] = jnp.zeros_like(res_scr)

    res = res_scr[...] + hid_scr[...]
    res_scr[...] = res
    x = _ln(res, lnw_ref[0], lnb_ref[0])

    xz = jnp.dot(x, WinT_ref[0], preferred_element_type=jnp.float32)
    xc0 = xz[:, :D_INNER]
    z = xz[:, D_INNER:]

    cw = cwT_ref[0]
    xp = jnp.concatenate([jnp.zeros((D_CONV - 1, D_INNER), jnp.float32), xc0],
                         axis=0)
    acc = xp[0:L, :] * cw[0:1, :]
    for k in range(1, D_CONV):
        acc = acc + xp[k:k + L, :] * cw[k:k + 1, :]
    xc = _silu(acc + cb_ref[0])

    proj = jnp.dot(xc, WxpT_ref[0], preferred_element_type=jnp.float32)
    dt = proj[:, :DT_RANK]
    Bs = proj[:, DT_RANK:DT_RANK + D_STATE]
    Cs = proj[:, DT_RANK + D_STATE:]

    delta = _softplus(
        jnp.dot(dt, WdtT_ref[0], preferred_element_type=jnp.float32)
        + bdt_ref[0])

    A = -jnp.exp(AT_ref[0])
    dA_scr[...] = jnp.exp(delta[:, None, :] * A[None, :, :])
    bu_scr[...] = (delta * xc)[:, None, :] * Bs[:, :, None]

    pltpu.semaphore_signal(fence_sem)
    pltpu.semaphore_wait(fence_sem, 1)

    def step(i, h):
        t = i * 4
        for k in range(4):
            h = dA_scr[t + k] * h + bu_scr[t + k]
            h_scr[t + k] = h
        return h

    jax.lax.fori_loop(0, L // 4, step,
                      jnp.zeros((D_STATE, D_INNER), jnp.float32))

    H = h_scr[...]
    ys = jnp.sum(H * Cs[:, :, None], axis=1)
    y = ys + xc * Dp_ref[0]
    y = y * _silu(z)
    out = jnp.dot(y, WoutT_ref[0], preferred_element_type=jnp.float32)
    hid_scr[...] = out

    @pl.when(d == DEPTH - 1)
    def _():
        resf = res + out
        hf = _ln(resf, nfw_ref[...], nfb_ref[...])
        pooled_ref[0] = jnp.mean(hf, axis=0, keepdims=True)


def _head_body(p_ref, w_ref, b_ref, o_ref):
    o_ref[...] = (jnp.dot(p_ref[...], w_ref[...],
                          preferred_element_type=jnp.float32) + b_ref[...])


def kernel(x, patch_w, patch_b, pos_embed, W_in, conv_w, conv_b, W_xp, W_dt,
           b_dt, A_log, D_p, W_out, ln_w, ln_b, normf_w, normf_b, head_w,
           head_b):
    f32 = jnp.float32
    patches = x.reshape(B, 3, NP, PATCH, NP, PATCH).transpose(0, 2, 4, 1, 3, 5)
    patches = patches.reshape(B, L, PDIM)
    pwT = patch_w.reshape(D_MODEL, PDIM).T
    pb = patch_b.reshape(1, D_MODEL)
    pos = pos_embed.reshape(L, D_MODEL)
    WinT = jnp.swapaxes(W_in, 1, 2)
    cwT = jnp.swapaxes(conv_w, 1, 2)
    cb = conv_b.reshape(DEPTH, 1, D_INNER)
    WxpT = jnp.swapaxes(W_xp, 1, 2)
    WdtT = jnp.swapaxes(W_dt, 1, 2)
    bdt = b_dt.reshape(DEPTH, 1, D_INNER)
    AT = jnp.swapaxes(A_log, 1, 2)
    Dp = D_p.reshape(DEPTH, 1, D_INNER)
    WoutT = jnp.swapaxes(W_out, 1, 2)
    lnw = ln_w.reshape(DEPTH, 1, D_MODEL)
    lnb = ln_b.reshape(DEPTH, 1, D_MODEL)
    nfw = normf_w.reshape(1, D_MODEL)
    nfb = normf_b.reshape(1, D_MODEL)

    bcast = lambda i, d: (0, 0)
    perb = lambda i, d: (i, 0, 0)
    perd = lambda i, d: (d, 0, 0)

    pooled = pl.pallas_call(
        _vim_body,
        grid=(B, DEPTH),
        in_specs=[
            pl.BlockSpec((1, L, PDIM), perb),
            pl.BlockSpec((PDIM, D_MODEL), bcast),
            pl.BlockSpec((1, D_MODEL), bcast),
            pl.BlockSpec((L, D_MODEL), bcast),
            pl.BlockSpec((1, D_MODEL, 2 * D_INNER), perd),
            pl.BlockSpec((1, D_CONV, D_INNER), perd),
            pl.BlockSpec((1, 1, D_INNER), perd),
            pl.BlockSpec((1, D_INNER, DT_RANK + 2 * D_STATE), perd),
            pl.BlockSpec((1, DT_RANK, D_INNER), perd),
            pl.BlockSpec((1, 1, D_INNER), perd),
            pl.BlockSpec((1, D_STATE, D_INNER), perd),
            pl.BlockSpec((1, 1, D_INNER), perd),
            pl.BlockSpec((1, D_INNER, D_MODEL), perd),
            pl.BlockSpec((1, 1, D_MODEL), perd),
            pl.BlockSpec((1, 1, D_MODEL), perd),
            pl.BlockSpec((1, D_MODEL), bcast),
            pl.BlockSpec((1, D_MODEL), bcast),
        ],
        out_specs=pl.BlockSpec((1, 1, D_MODEL), perb),
        out_shape=jax.ShapeDtypeStruct((B, 1, D_MODEL), f32),
        scratch_shapes=[
            pltpu.VMEM((L, D_MODEL), f32),
            pltpu.VMEM((L, D_MODEL), f32),
            pltpu.VMEM((L, D_STATE, D_INNER), f32),
            pltpu.VMEM((L, D_STATE, D_INNER), f32),
            pltpu.VMEM((L, D_STATE, D_INNER), f32),
            pltpu.SemaphoreType.REGULAR,
        ],
        compiler_params=pltpu.CompilerParams(
            dimension_semantics=("parallel", "arbitrary"),
        ),
    )(patches, pwT, pb, pos, WinT, cwT, cb, WxpT, WdtT, bdt, AT, Dp, WoutT,
      lnw, lnb, nfw, nfb)

    logits = pl.pallas_call(
        _head_body,
        out_shape=jax.ShapeDtypeStruct((B, NCLS), f32),
    )(pooled.reshape(B, D_MODEL), head_w.T, head_b.reshape(1, NCLS))
    return logits


# speedup vs baseline: 9.5724x; 1.1021x over previous
import jax
import jax.numpy as jnp
from jax.experimental import pallas as pl
from jax.experimental.pallas import tpu as pltpu

B, IMG, PATCH = 32, 224, 16
D_MODEL, DEPTH = 384, 24
D_INNER, D_STATE, DT_RANK, D_CONV = 768, 16, 24, 4
L = (IMG // PATCH) ** 2
NP = IMG // PATCH
NCLS = 1000
EPS = 1e-5
PDIM = 3 * PATCH * PATCH


def _ln(x, w, b):
    mu = jnp.mean(x, axis=-1, keepdims=True)
    xc = x - mu
    var = jnp.mean(xc * xc, axis=-1, keepdims=True)
    return xc * jax.lax.rsqrt(var + EPS) * w + b


def _silu(x):
    return x / (1.0 + jnp.exp(-x))


def _softplus(x):
    return jnp.maximum(x, 0.0) + jnp.log(1.0 + jnp.exp(-jnp.abs(x)))


def _vim_body(patches_ref, pwT_ref, pb_ref, pos_ref, WinT_ref, cwT_ref, cb_ref,
              WxpT_ref, WdtT_ref, bdt_ref, AT_ref, Dp_ref, WoutT_ref,
              lnw_ref, lnb_ref, nfw_ref, nfb_ref,
              pooled_ref,
              res_scr, hid_scr, dA_scr, bu_scr, h_scr):
    d = pl.program_id(1)

    @pl.when(d == 0)
    def _():
        p = patches_ref[0]
        h0 = jnp.dot(p, pwT_ref[...], preferred_element_type=jnp.float32)
        hid_scr[...] = h0 + pb_ref[...] + pos_ref[...]
        res_scr[...] = jnp.zeros_like(res_scr)

    res = res_scr[...] + hid_scr[...]
    res_scr[...] = res
    x = _ln(res, lnw_ref[0], lnb_ref[0])

    xz = jnp.dot(x, WinT_ref[0], preferred_element_type=jnp.float32)
    xc0 = xz[:, :D_INNER]
    z = xz[:, D_INNER:]

    cw = cwT_ref[0]
    xp = jnp.concatenate([jnp.zeros((D_CONV - 1, D_INNER), jnp.float32), xc0],
                         axis=0)
    acc = xp[0:L, :] * cw[0:1, :]
    for k in range(1, D_CONV):
        acc = acc + xp[k:k + L, :] * cw[k:k + 1, :]
    xc = _silu(acc + cb_ref[0])

    proj = jnp.dot(xc, WxpT_ref[0], preferred_element_type=jnp.float32)
    dt = proj[:, :DT_RANK]
    Bs = proj[:, DT_RANK:DT_RANK + D_STATE]
    Cs = proj[:, DT_RANK + D_STATE:]

    delta = _softplus(
        jnp.dot(dt, WdtT_ref[0], preferred_element_type=jnp.float32)
        + bdt_ref[0])

    A = -jnp.exp(AT_ref[0])
    dA_scr[...] = jnp.exp(delta[:, None, :] * A[None, :, :])
    bu_scr[...] = (delta * xc)[:, None, :] * Bs[:, :, None]

    h = jnp.zeros((D_STATE, D_INNER), jnp.float32)
    for t in range(L):
        h = dA_scr[t] * h + bu_scr[t]
        h_scr[t] = h

    H = h_scr[...]
    ys = jnp.sum(H * Cs[:, :, None], axis=1)
    y = ys + xc * Dp_ref[0]
    y = y * _silu(z)
    out = jnp.dot(y, WoutT_ref[0], preferred_element_type=jnp.float32)
    hid_scr[...] = out

    @pl.when(d == DEPTH - 1)
    def _():
        resf = res + out
        hf = _ln(resf, nfw_ref[...], nfb_ref[...])
        pooled_ref[0] = jnp.mean(hf, axis=0, keepdims=True)


def _head_body(p_ref, w_ref, b_ref, o_ref):
    o_ref[...] = (jnp.dot(p_ref[...], w_ref[...],
                          preferred_element_type=jnp.float32) + b_ref[...])


def kernel(x, patch_w, patch_b, pos_embed, W_in, conv_w, conv_b, W_xp, W_dt,
           b_dt, A_log, D_p, W_out, ln_w, ln_b, normf_w, normf_b, head_w,
           head_b):
    f32 = jnp.float32
    patches = x.reshape(B, 3, NP, PATCH, NP, PATCH).transpose(0, 2, 4, 1, 3, 5)
    patches = patches.reshape(B, L, PDIM)
    pwT = patch_w.reshape(D_MODEL, PDIM).T
    pb = patch_b.reshape(1, D_MODEL)
    pos = pos_embed.reshape(L, D_MODEL)
    WinT = jnp.swapaxes(W_in, 1, 2)
    cwT = jnp.swapaxes(conv_w, 1, 2)
    cb = conv_b.reshape(DEPTH, 1, D_INNER)
    WxpT = jnp.swapaxes(W_xp, 1, 2)
    WdtT = jnp.swapaxes(W_dt, 1, 2)
    bdt = b_dt.reshape(DEPTH, 1, D_INNER)
    AT = jnp.swapaxes(A_log, 1, 2)
    Dp = D_p.reshape(DEPTH, 1, D_INNER)
    WoutT = jnp.swapaxes(W_out, 1, 2)
    lnw = ln_w.reshape(DEPTH, 1, D_MODEL)
    lnb = ln_b.reshape(DEPTH, 1, D_MODEL)
    nfw = normf_w.reshape(1, D_MODEL)
    nfb = normf_b.reshape(1, D_MODEL)

    bcast = lambda i, d: (0, 0)
    perb = lambda i, d: (i, 0, 0)
    perd = lambda i, d: (d, 0, 0)

    pooled = pl.pallas_call(
        _vim_body,
        grid=(B, DEPTH),
        in_specs=[
            pl.BlockSpec((1, L, PDIM), perb),
            pl.BlockSpec((PDIM, D_MODEL), bcast),
            pl.BlockSpec((1, D_MODEL), bcast),
            pl.BlockSpec((L, D_MODEL), bcast),
            pl.BlockSpec((1, D_MODEL, 2 * D_INNER), perd),
            pl.BlockSpec((1, D_CONV, D_INNER), perd),
            pl.BlockSpec((1, 1, D_INNER), perd),
            pl.BlockSpec((1, D_INNER, DT_RANK + 2 * D_STATE), perd),
            pl.BlockSpec((1, DT_RANK, D_INNER), perd),
            pl.BlockSpec((1, 1, D_INNER), perd),
            pl.BlockSpec((1, D_STATE, D_INNER), perd),
            pl.BlockSpec((1, 1, D_INNER), perd),
            pl.BlockSpec((1, D_INNER, D_MODEL), perd),
            pl.BlockSpec((1, 1, D_MODEL), perd),
            pl.BlockSpec((1, 1, D_MODEL), perd),
            pl.BlockSpec((1, D_MODEL), bcast),
            pl.BlockSpec((1, D_MODEL), bcast),
        ],
        out_specs=pl.BlockSpec((1, 1, D_MODEL), perb),
        out_shape=jax.ShapeDtypeStruct((B, 1, D_MODEL), f32),
        scratch_shapes=[
            pltpu.VMEM((L, D_MODEL), f32),
            pltpu.VMEM((L, D_MODEL), f32),
            pltpu.VMEM((L, D_STATE, D_INNER), f32),
            pltpu.VMEM((L, D_STATE, D_INNER), f32),
            pltpu.VMEM((L, D_STATE, D_INNER), f32),
        ],
        compiler_params=pltpu.CompilerParams(
            dimension_semantics=("parallel", "arbitrary"),
        ),
    )(patches, pwT, pb, pos, WinT, cwT, cb, WxpT, WdtT, bdt, AT, Dp, WoutT,
      lnw, lnb, nfw, nfb)

    logits = pl.pallas_call(
        _head_body,
        out_shape=jax.ShapeDtypeStruct((B, NCLS), f32),
    )(pooled.reshape(B, D_MODEL), head_w.T, head_b.reshape(1, NCLS))
    return logits
